# Initial kernel scaffold; baseline (speedup 1.0000x reference)
import jax
import jax.numpy as jnp
from jax.experimental import pallas as pl


def kernel(x, edge_index, W, b, gamma, beta):
    raise NotImplementedError("write your pallas kernel here")


# speedup vs baseline: 4.1187x; 4.1187x over previous
import functools

import jax
import jax.numpy as jnp
from jax import lax
from jax.experimental import pallas as pl
from jax.experimental.pallas import tpu as pltpu
from jax.experimental.pallas import tpu_sc as plsc

N_NODES = 10000
N_EDGES = 160000
D = 256
L = 128
NC = 2
NS = 16
B = 128
NBT = 80
EP = NS * NBT * B
ROWS = 10112
RPT = ROWS // NS
TRASH = 10104


def _sc_aggregate(x2, gidx, dstb):
    mesh = plsc.VectorSubcoreMesh(core_axis_name="c", subcore_axis_name="s")

    @functools.partial(
        pl.kernel,
        out_type=[
            jax.ShapeDtypeStruct((NC, ROWS, L), jnp.float32),
            jax.ShapeDtypeStruct((ROWS,), jnp.float32),
        ],
        mesh=mesh,
        scratch_types=[
            pltpu.VMEM((B,), jnp.int32),
            pltpu.VMEM((B,), jnp.int32),
            pltpu.VMEM((B,), jnp.int32),
            pltpu.VMEM((B,), jnp.float32),
            pltpu.VMEM((RPT,), jnp.float32),
            pltpu.VMEM((B, L), jnp.float32),
            pltpu.VMEM((B, L), jnp.float32),
            pltpu.VMEM_SHARED((ROWS, L), jnp.float32),
            pltpu.VMEM_SHARED((ROWS,), jnp.float32),
            pltpu.SemaphoreType.DMA,
            pltpu.SemaphoreType.DMA,
        ],
    )
    def sc_kernel(x2_h, gidx_h, dstb_h, agg_o, deg_o,
                  gib0, gib1, dib, ones1, dstg, rows0, rows1, acc_s, deg_s,
                  sem0, sem1):
        c = lax.axis_index("c")
        s = lax.axis_index("s")
        base = s * RPT
        zero = jnp.zeros((16,), jnp.float32)
        one = jnp.full((16,), 1.0, jnp.float32)

        def zr(i, _):
            def zc(j, _):
                rows0[i, pl.ds(pl.multiple_of(j * 16, 16), 16)] = zero
                return 0
            lax.fori_loop(0, L // 16, zc, 0)
            return 0
        lax.fori_loop(0, B, zr, 0)

        def orow(j, _):
            ones1[pl.ds(pl.multiple_of(j * 16, 16), 16)] = one
            return 0
        lax.fori_loop(0, B // 16, orow, 0)

        def zdg(j, _):
            dstg[pl.ds(pl.multiple_of(j * 16, 16), 16)] = zero
            return 0
        lax.fori_loop(0, RPT // 16, zdg, 0)
        if RPT % 16:
            dstg[pl.ds(RPT - 16, 16)] = zero

        nfull, rem = RPT // B, RPT % B
        for k in range(nfull):
            pltpu.sync_copy(rows0, acc_s.at[pl.ds(base + k * B, B)])
        if rem:
            pltpu.sync_copy(rows0.at[pl.ds(0, rem)],
                            acc_s.at[pl.ds(base + nfull * B, rem)])

        @pl.when(c == 0)
        def _():
            pltpu.sync_copy(dstg, deg_s.at[pl.ds(base, RPT)])

        plsc.subcore_barrier()

        grow = (c * NS + s) * NBT
        drow = s * NBT

        def scat(t, buf):
            pltpu.sync_copy(dstb_h.at[drow + t], dib)
            pltpu.sync_copy(buf, acc_s.at[dib], add=True)

            @pl.when(c == 0)
            def _():
                pltpu.sync_copy(ones1, deg_s.at[dib], add=True)

        bufs = ((gib0, rows0, sem0), (gib1, rows1, sem1))
        for par in (0, 1):
            gb, rv, sm = bufs[par]
            pltpu.sync_copy(gidx_h.at[grow + par], gb)
            pltpu.async_copy(x2_h.at[gb], rv, sm)

        def pair(t2, _):
            for par in (0, 1):
                gb, rv, sm = bufs[par]
                t = t2 * 2 + par
                pltpu.make_async_copy(x2_h.at[gb], rv, sm).wait()
                scat(t, rv)

                @pl.when(t + 2 < NBT)
                def _():
                    pltpu.sync_copy(gidx_h.at[grow + t + 2], gb)
                    pltpu.async_copy(x2_h.at[gb], rv, sm)
            return 0
        lax.fori_loop(0, NBT // 2, pair, 0)

        plsc.subcore_barrier()

        pltpu.sync_copy(acc_s.at[pl.ds(base, RPT)],
                        agg_o.at[c].at[pl.ds(base, RPT)])

        @pl.when(c == 0)
        def _():
            pltpu.sync_copy(deg_s.at[pl.ds(base, RPT)], dstg)
            pltpu.sync_copy(dstg, deg_o.at[pl.ds(base, RPT)])

    return sc_kernel(x2, gidx, dstb)


def _tc_block(agg_ref, dg_ref, w_ref, b_ref, g_ref, be_ref, o_ref):
    inv = 1.0 / jnp.maximum(dg_ref[...], 1.0)
    a0 = agg_ref[0] * inv
    a1 = agg_ref[1] * inv
    h = jnp.dot(a0, w_ref[0:L, :], preferred_element_type=jnp.float32)
    h = h + jnp.dot(a1, w_ref[L:D, :], preferred_element_type=jnp.float32)
    h = h + b_ref[...]
    mu = jnp.mean(h, axis=-1, keepdims=True)
    d = h - mu
    var = jnp.mean(d * d, axis=-1, keepdims=True)
    h = d * lax.rsqrt(var + 1e-5) * g_ref[...] + be_ref[...]
    o_ref[...] = jnp.maximum(h, 0.0)


def kernel(x, edge_index, W, b, gamma, beta):
    src = edge_index[0]
    dst = edge_index[1]
    pad = EP - N_EDGES
    src_p = jnp.concatenate([src, jnp.zeros((pad,), jnp.int32)])
    dst_p = jnp.concatenate([dst, jnp.full((pad,), TRASH, jnp.int32)])
    gidx = (src_p * 2)[None, :] + jnp.arange(NC, dtype=jnp.int32)[:, None]
    gidx = gidx.reshape(NC * NS * NBT, B)
    dstb = dst_p.reshape(NS * NBT, B)
    x2 = x.reshape(2 * N_NODES, L)

    agg, deg = _sc_aggregate(x2, gidx, dstb)

    out = pl.pallas_call(
        _tc_block,
        grid=(NS,),
        in_specs=[
            pl.BlockSpec((NC, RPT, L), lambda i: (0, i, 0)),
            pl.BlockSpec((RPT, 1), lambda i: (i, 0)),
            pl.BlockSpec((D, D), lambda i: (0, 0)),
            pl.BlockSpec((1, D), lambda i: (0, 0)),
            pl.BlockSpec((1, D), lambda i: (0, 0)),
            pl.BlockSpec((1, D), lambda i: (0, 0)),
        ],
        out_specs=pl.BlockSpec((RPT, D), lambda i: (i, 0)),
        out_shape=jax.ShapeDtypeStruct((ROWS, D), jnp.float32),
    )(agg, deg.reshape(ROWS, 1), W,
      b.reshape(1, D), gamma.reshape(1, D), beta.reshape(1, D))

    return out[:N_NODES]
